# baseline (speedup 1.0000x reference)
import dataclasses
import functools

import jax
import jax.numpy as jnp
from jax import lax
from jax.experimental import pallas as pl
from jax.experimental.pallas import tpu as pltpu
from jax.experimental.pallas import tpu_sc as plsc

N, D, H, E = 16384, 4096, 128, 64
TOPK = 8
TILE_N = 512
CHUNK_SIZES = (8192, 6144, 2048)
NWORK = 32


def _gemm_body(t_ref, x_ref, w1_ref, b1_ref, w2_ref, b2_ref, logits_ref):
    t = jnp.clip(t_ref[0], 0.5, 2.0)
    h = jnp.dot(x_ref[...], w1_ref[...], preferred_element_type=jnp.float32)
    h = jax.nn.relu(h + b1_ref[...])
    logits = jnp.dot(h, w2_ref[...], preferred_element_type=jnp.float32)
    logits_ref[...] = (logits + b2_ref[...]) / t


def _tc_logits(temperature, x, W1, b1, W2, b2, start, size):
    return pl.pallas_call(
        _gemm_body,
        grid=(size // TILE_N,),
        in_specs=[
            pl.BlockSpec(memory_space=pltpu.SMEM),
            pl.BlockSpec((TILE_N, D),
                         lambda i, s=start: (s // TILE_N + i, 0)),
            pl.BlockSpec((D, H), lambda i: (0, 0)),
            pl.BlockSpec((1, H), lambda i: (0, 0)),
            pl.BlockSpec((H, E), lambda i: (0, 0)),
            pl.BlockSpec((1, E), lambda i: (0, 0)),
        ],
        out_specs=pl.BlockSpec((TILE_N, E), lambda i: (i, 0)),
        out_shape=jax.ShapeDtypeStruct((size, E), jnp.float32),
    )(temperature, x, W1, b1.reshape(1, H), W2, b2.reshape(1, E))


def _sc_route_body(RW, logits_hbm, gates_hbm, idx_hbm, logits_v, gates_v,
                   idx_v):
    wid = lax.axis_index("s") * 2 + lax.axis_index("c")
    base = wid * RW
    pltpu.sync_copy(logits_hbm.at[pl.ds(base, RW)], logits_v)

    iota16 = lax.broadcasted_iota(jnp.int32, (16,), 0)
    lane_lt8 = iota16 < 8
    zeros16 = jnp.zeros((16,), jnp.float32)

    def merge(av, ai, bv, bi):
        cv = jnp.where(lane_lt8, av, lax.rev(bv, (0,)))
        ci = jnp.where(lane_lt8, ai, lax.rev(bi, (0,)))
        return plsc.sort_key_val(cv, ci, descending=True)

    @plsc.parallel_loop(0, RW, unroll=4)
    def _(r):
        sv = []
        si = []
        for j in range(4):
            v = logits_v[r, pl.ds(16 * j, 16)]
            skv, ski = plsc.sort_key_val(v, iota16 + 16 * j, descending=True)
            sv.append(skv)
            si.append(ski)
        m0v, m0i = merge(sv[0], si[0], sv[1], si[1])
        m1v, m1i = merge(sv[2], si[2], sv[3], si[3])
        tv, ti = merge(m0v, m0i, m1v, m1i)

        mx = jnp.max(tv)
        e8 = jnp.where(lane_lt8, jnp.exp(tv - mx), 0.0)
        g = e8 / jnp.sum(e8)

        for j in range(4):
            gates_v[r, pl.ds(16 * j, 16)] = zeros16
        row = jnp.full((16,), r, jnp.int32)
        plsc.store_scatter(gates_v, [row, ti], g, mask=lane_lt8)
        idx_v[r, :] = ti

    pltpu.sync_copy(gates_v, gates_hbm.at[pl.ds(base, RW)])
    pltpu.sync_copy(idx_v, idx_hbm.at[pl.ds(base, RW)])


def _sc_route(logits, size):
    rw = size // NWORK
    mesh = plsc.VectorSubcoreMesh(core_axis_name="c", subcore_axis_name="s")
    out_type = (
        jax.ShapeDtypeStruct((size, E), jnp.float32),
        jax.ShapeDtypeStruct((size, 16), jnp.int32),
    )
    scratch = [
        pltpu.VMEM((rw, E), jnp.float32),
        pltpu.VMEM((rw, E), jnp.float32),
        pltpu.VMEM((rw, 16), jnp.int32),
    ]
    cp = pltpu.CompilerParams()
    if "needs_layout_passes" in pltpu.CompilerParams.__dataclass_fields__:
        cp = dataclasses.replace(cp, needs_layout_passes=False)
    return pl.kernel(functools.partial(_sc_route_body, rw), mesh=mesh,
                     out_type=out_type, scratch_types=scratch,
                     compiler_params=cp)(logits)


@jax.jit
def kernel(x, W1, b1, W2, b2, temperature):
    nchunk = len(CHUNK_SIZES)
    starts = [sum(CHUNK_SIZES[:c]) for c in range(nchunk)]
    logits_c = [None] * nchunk
    gates_c = [None] * nchunk
    idx_c = [None] * nchunk
    logits_c[0] = _tc_logits(temperature, x, W1, b1, W2, b2, starts[0],
                             CHUNK_SIZES[0])
    for c in range(nchunk):
        gates_c[c], idx_c[c] = _sc_route(logits_c[c], CHUNK_SIZES[c])
        if c + 1 < nchunk:
            logits_c[c + 1] = _tc_logits(temperature, x, W1, b1, W2, b2,
                                         starts[c + 1], CHUNK_SIZES[c + 1])
    logits = jnp.concatenate(logits_c, axis=0)
    gates = jnp.concatenate(gates_c, axis=0)
    idx = jnp.concatenate(idx_c, axis=0)[:, :TOPK]
    return (gates, idx, logits)


# speedup vs baseline: 1.0224x; 1.0224x over previous
import dataclasses
import functools

import jax
import jax.numpy as jnp
from jax import lax
from jax.experimental import pallas as pl
from jax.experimental.pallas import tpu as pltpu
from jax.experimental.pallas import tpu_sc as plsc

N, D, H, E = 16384, 4096, 128, 64
TOPK = 8
TILE_N = 512
CHUNK_SIZES = (10240, 6144)
NWORK = 32


def _gemm_body(t_ref, x_ref, w1_ref, b1_ref, w2_ref, b2_ref, logits_ref):
    t = jnp.clip(t_ref[0], 0.5, 2.0)
    h = jnp.dot(x_ref[...], w1_ref[...], preferred_element_type=jnp.float32)
    h = jax.nn.relu(h + b1_ref[...])
    logits = jnp.dot(h, w2_ref[...], preferred_element_type=jnp.float32)
    logits_ref[...] = (logits + b2_ref[...]) / t


def _tc_logits(temperature, x, W1, b1, W2, b2, start, size):
    return pl.pallas_call(
        _gemm_body,
        grid=(size // TILE_N,),
        in_specs=[
            pl.BlockSpec(memory_space=pltpu.SMEM),
            pl.BlockSpec((TILE_N, D),
                         lambda i, s=start: (s // TILE_N + i, 0)),
            pl.BlockSpec((D, H), lambda i: (0, 0)),
            pl.BlockSpec((1, H), lambda i: (0, 0)),
            pl.BlockSpec((H, E), lambda i: (0, 0)),
            pl.BlockSpec((1, E), lambda i: (0, 0)),
        ],
        out_specs=pl.BlockSpec((TILE_N, E), lambda i: (i, 0)),
        out_shape=jax.ShapeDtypeStruct((size, E), jnp.float32),
    )(temperature, x, W1, b1.reshape(1, H), W2, b2.reshape(1, E))


def _sc_route_body(RW, logits_hbm, gates_hbm, idx_hbm, logits_v, gates_v,
                   idx_v):
    wid = lax.axis_index("s") * 2 + lax.axis_index("c")
    base = wid * RW
    pltpu.sync_copy(logits_hbm.at[pl.ds(base, RW)], logits_v)

    iota16 = lax.broadcasted_iota(jnp.int32, (16,), 0)
    lane_lt8 = iota16 < 8
    zeros16 = jnp.zeros((16,), jnp.float32)

    def merge(av, ai, bv, bi):
        cv = jnp.where(lane_lt8, av, lax.rev(bv, (0,)))
        ci = jnp.where(lane_lt8, ai, lax.rev(bi, (0,)))
        return plsc.sort_key_val(cv, ci, descending=True)

    @plsc.parallel_loop(0, RW, unroll=4)
    def _(r):
        sv = []
        si = []
        for j in range(4):
            v = logits_v[r, pl.ds(16 * j, 16)]
            skv, ski = plsc.sort_key_val(v, iota16 + 16 * j, descending=True)
            sv.append(skv)
            si.append(ski)
        m0v, m0i = merge(sv[0], si[0], sv[1], si[1])
        m1v, m1i = merge(sv[2], si[2], sv[3], si[3])
        tv, ti = merge(m0v, m0i, m1v, m1i)

        mx = jnp.max(tv)
        e8 = jnp.where(lane_lt8, jnp.exp(tv - mx), 0.0)
        g = e8 / jnp.sum(e8)

        for j in range(4):
            gates_v[r, pl.ds(16 * j, 16)] = zeros16
        row = jnp.full((16,), r, jnp.int32)
        plsc.store_scatter(gates_v, [row, ti], g, mask=lane_lt8)
        idx_v[r, :] = ti

    pltpu.sync_copy(gates_v, gates_hbm.at[pl.ds(base, RW)])
    pltpu.sync_copy(idx_v, idx_hbm.at[pl.ds(base, RW)])


def _sc_route(logits, size):
    rw = size // NWORK
    mesh = plsc.VectorSubcoreMesh(core_axis_name="c", subcore_axis_name="s")
    out_type = (
        jax.ShapeDtypeStruct((size, E), jnp.float32),
        jax.ShapeDtypeStruct((size, 16), jnp.int32),
    )
    scratch = [
        pltpu.VMEM((rw, E), jnp.float32),
        pltpu.VMEM((rw, E), jnp.float32),
        pltpu.VMEM((rw, 16), jnp.int32),
    ]
    cp = pltpu.CompilerParams()
    if "needs_layout_passes" in pltpu.CompilerParams.__dataclass_fields__:
        cp = dataclasses.replace(cp, needs_layout_passes=False)
    return pl.kernel(functools.partial(_sc_route_body, rw), mesh=mesh,
                     out_type=out_type, scratch_types=scratch,
                     compiler_params=cp)(logits)


@jax.jit
def kernel(x, W1, b1, W2, b2, temperature):
    nchunk = len(CHUNK_SIZES)
    starts = [sum(CHUNK_SIZES[:c]) for c in range(nchunk)]
    logits_c = [None] * nchunk
    gates_c = [None] * nchunk
    idx_c = [None] * nchunk
    logits_c[0] = _tc_logits(temperature, x, W1, b1, W2, b2, starts[0],
                             CHUNK_SIZES[0])
    for c in range(nchunk):
        gates_c[c], idx_c[c] = _sc_route(logits_c[c], CHUNK_SIZES[c])
        if c + 1 < nchunk:
            logits_c[c + 1] = _tc_logits(temperature, x, W1, b1, W2, b2,
                                         starts[c + 1], CHUNK_SIZES[c + 1])
    logits = jnp.concatenate(logits_c, axis=0)
    gates = jnp.concatenate(gates_c, axis=0)
    idx = jnp.concatenate(idx_c, axis=0)[:, :TOPK]
    return (gates, idx, logits)


# speedup vs baseline: 1.0296x; 1.0070x over previous
import dataclasses
import functools

import jax
import jax.numpy as jnp
from jax import lax
from jax.experimental import pallas as pl
from jax.experimental.pallas import tpu as pltpu
from jax.experimental.pallas import tpu_sc as plsc

N, D, H, E = 16384, 4096, 128, 64
TOPK = 8
TILE_N = 1024
NCHUNK = 2
CHUNK = N // NCHUNK
NWORK = 32
RW = CHUNK // NWORK


def _gemm_body(t_ref, x_ref, w1_ref, b1_ref, w2_ref, b2_ref, logits_ref):
    t = jnp.clip(t_ref[0], 0.5, 2.0)
    h = jnp.dot(x_ref[...], w1_ref[...], preferred_element_type=jnp.float32)
    h = jax.nn.relu(h + b1_ref[...])
    logits = jnp.dot(h, w2_ref[...], preferred_element_type=jnp.float32)
    logits_ref[...] = (logits + b2_ref[...]) / t


def _tc_logits(temperature, x, W1, b1, W2, b2, chunk):
    return pl.pallas_call(
        _gemm_body,
        grid=(CHUNK // TILE_N,),
        in_specs=[
            pl.BlockSpec(memory_space=pltpu.SMEM),
            pl.BlockSpec((TILE_N, D),
                         lambda i, c=chunk: (c * (CHUNK // TILE_N) + i, 0)),
            pl.BlockSpec((D, H), lambda i: (0, 0)),
            pl.BlockSpec((1, H), lambda i: (0, 0)),
            pl.BlockSpec((H, E), lambda i: (0, 0)),
            pl.BlockSpec((1, E), lambda i: (0, 0)),
        ],
        out_specs=pl.BlockSpec((TILE_N, E), lambda i: (i, 0)),
        out_shape=jax.ShapeDtypeStruct((CHUNK, E), jnp.float32),
    )(temperature, x, W1, b1.reshape(1, H), W2, b2.reshape(1, E))


def _sc_route_body(logits_hbm, gates_hbm, idx_hbm, logits_v, gates_v, idx_v):
    wid = lax.axis_index("s") * 2 + lax.axis_index("c")
    base = wid * RW
    pltpu.sync_copy(logits_hbm.at[pl.ds(base, RW)], logits_v)

    iota16 = lax.broadcasted_iota(jnp.int32, (16,), 0)
    lane_lt8 = iota16 < 8
    zeros16 = jnp.zeros((16,), jnp.float32)

    def merge(av, ai, bv, bi):
        cv = jnp.where(lane_lt8, av, lax.rev(bv, (0,)))
        ci = jnp.where(lane_lt8, ai, lax.rev(bi, (0,)))
        return plsc.sort_key_val(cv, ci, descending=True)

    @plsc.parallel_loop(0, RW, unroll=4)
    def _(r):
        sv = []
        si = []
        for j in range(4):
            v = logits_v[r, pl.ds(16 * j, 16)]
            skv, ski = plsc.sort_key_val(v, iota16 + 16 * j, descending=True)
            sv.append(skv)
            si.append(ski)
        m0v, m0i = merge(sv[0], si[0], sv[1], si[1])
        m1v, m1i = merge(sv[2], si[2], sv[3], si[3])
        tv, ti = merge(m0v, m0i, m1v, m1i)

        mx = jnp.max(tv)
        e8 = jnp.where(lane_lt8, jnp.exp(tv - mx), 0.0)
        g = e8 / jnp.sum(e8)

        for j in range(4):
            gates_v[r, pl.ds(16 * j, 16)] = zeros16
        row = jnp.full((16,), r, jnp.int32)
        plsc.store_scatter(gates_v, [row, ti], g, mask=lane_lt8)
        idx_v[r, :] = ti

    pltpu.sync_copy(gates_v, gates_hbm.at[pl.ds(base, RW)])
    pltpu.sync_copy(idx_v, idx_hbm.at[pl.ds(base, RW)])


def _sc_route(logits):
    mesh = plsc.VectorSubcoreMesh(core_axis_name="c", subcore_axis_name="s")
    out_type = (
        jax.ShapeDtypeStruct((CHUNK, E), jnp.float32),
        jax.ShapeDtypeStruct((CHUNK, 16), jnp.int32),
    )
    scratch = [
        pltpu.VMEM((RW, E), jnp.float32),
        pltpu.VMEM((RW, E), jnp.float32),
        pltpu.VMEM((RW, 16), jnp.int32),
    ]
    cp = pltpu.CompilerParams()
    if "needs_layout_passes" in pltpu.CompilerParams.__dataclass_fields__:
        cp = dataclasses.replace(cp, needs_layout_passes=False)
    return pl.kernel(_sc_route_body, mesh=mesh, out_type=out_type,
                     scratch_types=scratch, compiler_params=cp)(logits)


@jax.jit
def kernel(x, W1, b1, W2, b2, temperature):
    logits_c = [None] * NCHUNK
    gates_c = [None] * NCHUNK
    idx_c = [None] * NCHUNK
    logits_c[0] = _tc_logits(temperature, x, W1, b1, W2, b2, 0)
    for c in range(NCHUNK):
        gates_c[c], idx_c[c] = _sc_route(logits_c[c])
        if c + 1 < NCHUNK:
            logits_c[c + 1] = _tc_logits(temperature, x, W1, b1, W2, b2, c + 1)
    logits = jnp.concatenate(logits_c, axis=0)
    gates = jnp.concatenate(gates_c, axis=0)
    idx = jnp.concatenate(idx_c, axis=0)[:, :TOPK]
    return (gates, idx, logits)


# speedup vs baseline: 1.0377x; 1.0080x over previous
import dataclasses
import functools

import jax
import jax.numpy as jnp
from jax import lax
from jax.experimental import pallas as pl
from jax.experimental.pallas import tpu as pltpu
from jax.experimental.pallas import tpu_sc as plsc

N, D, H, E = 16384, 4096, 128, 64
TOPK = 8
TILE_N = 512
NCHUNK = 2
CHUNK = N // NCHUNK
NWORK = 32
RW = CHUNK // NWORK


def _gemm_body(t_ref, x_ref, w1_ref, b1_ref, w2_ref, b2_ref, logits_ref):
    t = jnp.clip(t_ref[0], 0.5, 2.0)
    h = jnp.dot(x_ref[...], w1_ref[...], preferred_element_type=jnp.float32)
    h = jax.nn.relu(h + b1_ref[...])
    logits = jnp.dot(h, w2_ref[...], preferred_element_type=jnp.float32)
    logits_ref[...] = (logits + b2_ref[...]) / t


def _tc_logits(temperature, x, W1, b1, W2, b2, chunk):
    return pl.pallas_call(
        _gemm_body,
        grid=(CHUNK // TILE_N,),
        in_specs=[
            pl.BlockSpec(memory_space=pltpu.SMEM),
            pl.BlockSpec((TILE_N, D),
                         lambda i, c=chunk: (c * (CHUNK // TILE_N) + i, 0)),
            pl.BlockSpec((D, H), lambda i: (0, 0)),
            pl.BlockSpec((1, H), lambda i: (0, 0)),
            pl.BlockSpec((H, E), lambda i: (0, 0)),
            pl.BlockSpec((1, E), lambda i: (0, 0)),
        ],
        out_specs=pl.BlockSpec((TILE_N, E), lambda i: (i, 0)),
        out_shape=jax.ShapeDtypeStruct((CHUNK, E), jnp.float32),
    )(temperature, x, W1, b1.reshape(1, H), W2, b2.reshape(1, E))


def _sc_route_body(logits_hbm, gates_hbm, idx_hbm, logits_v, gates_v, idx_v):
    wid = lax.axis_index("s") * 2 + lax.axis_index("c")
    base = wid * RW
    pltpu.sync_copy(logits_hbm.at[pl.ds(base, RW)], logits_v)

    iota16 = lax.broadcasted_iota(jnp.int32, (16,), 0)
    lane_lt8 = iota16 < 8
    zeros16 = jnp.zeros((16,), jnp.float32)

    def merge(av, ai, bv, bi):
        cv = jnp.where(lane_lt8, av, lax.rev(bv, (0,)))
        ci = jnp.where(lane_lt8, ai, lax.rev(bi, (0,)))
        return plsc.sort_key_val(cv, ci, descending=True)

    @plsc.parallel_loop(0, RW, unroll=4)
    def _(r):
        sv = []
        si = []
        for j in range(4):
            v = logits_v[r, pl.ds(16 * j, 16)]
            skv, ski = plsc.sort_key_val(v, iota16 + 16 * j, descending=True)
            sv.append(skv)
            si.append(ski)
        m0v, m0i = merge(sv[0], si[0], sv[1], si[1])
        m1v, m1i = merge(sv[2], si[2], sv[3], si[3])
        tv, ti = merge(m0v, m0i, m1v, m1i)

        mx = jnp.max(tv)
        e8 = jnp.where(lane_lt8, jnp.exp(tv - mx), 0.0)
        g = e8 / jnp.sum(e8)

        for j in range(4):
            gates_v[r, pl.ds(16 * j, 16)] = zeros16
        row = jnp.full((16,), r, jnp.int32)
        plsc.store_scatter(gates_v, [row, ti], g, mask=lane_lt8)
        idx_v[r, :] = ti

    pltpu.sync_copy(gates_v, gates_hbm.at[pl.ds(base, RW)])
    pltpu.sync_copy(idx_v, idx_hbm.at[pl.ds(base, RW)])


def _sc_route(logits):
    mesh = plsc.VectorSubcoreMesh(core_axis_name="c", subcore_axis_name="s")
    out_type = (
        jax.ShapeDtypeStruct((CHUNK, E), jnp.float32),
        jax.ShapeDtypeStruct((CHUNK, 16), jnp.int32),
    )
    scratch = [
        pltpu.VMEM((RW, E), jnp.float32),
        pltpu.VMEM((RW, E), jnp.float32),
        pltpu.VMEM((RW, 16), jnp.int32),
    ]
    cp = pltpu.CompilerParams()
    if "needs_layout_passes" in pltpu.CompilerParams.__dataclass_fields__:
        cp = dataclasses.replace(cp, needs_layout_passes=False)
    return pl.kernel(_sc_route_body, mesh=mesh, out_type=out_type,
                     scratch_types=scratch, compiler_params=cp)(logits)


@jax.jit
def kernel(x, W1, b1, W2, b2, temperature):
    logits_c = [None] * NCHUNK
    gates_c = [None] * NCHUNK
    idx_c = [None] * NCHUNK
    logits_c[0] = _tc_logits(temperature, x, W1, b1, W2, b2, 0)
    for c in range(NCHUNK):
        gates_c[c], idx_c[c] = _sc_route(logits_c[c])
        if c + 1 < NCHUNK:
            logits_c[c + 1] = _tc_logits(temperature, x, W1, b1, W2, b2, c + 1)
    logits = jnp.concatenate(logits_c, axis=0)
    gates = jnp.concatenate(gates_c, axis=0)
    idx = jnp.concatenate(idx_c, axis=0)[:, :TOPK]
    return (gates, idx, logits)
